# baseline (speedup 1.0000x reference)
import functools

import jax
import jax.numpy as jnp
from jax import lax
from jax.experimental import pallas as pl
from jax.experimental.pallas import tpu as pltpu
from jax.experimental.pallas import tpu_sc as plsc

VOCAB = 1000000
D = 64
TOTAL = 4096 * 200
L = 16

NC = 2
NS = 16
NW = NC * NS
PER_W = TOTAL // NW
CHUNK = 512
N_CHUNKS = PER_W // CHUNK


def _body(idx_hbm, table_hbm, out_hbm, idx_v, rows_v, gsem):
    wid = lax.axis_index("s") * NC + lax.axis_index("c")
    wbase = wid * PER_W

    def chunk_step(i, carry):
        base = wbase + i * CHUNK
        pltpu.sync_copy(idx_hbm.at[pl.ds(base, CHUNK)], idx_v)
        gather = pltpu.async_copy(table_hbm.at[idx_v], rows_v, gsem)

        def min_step(g, acc):
            return jnp.minimum(acc, idx_v[pl.ds(g * L, L)])

        acc = lax.fori_loop(0, CHUNK // L, min_step,
                            jnp.full((L,), jnp.iinfo(jnp.int32).max, jnp.int32))
        chunk_min = acc[0]
        for g in range(1, L):
            chunk_min = jnp.minimum(chunk_min, acc[g])

        gather.wait()

        @pl.when(chunk_min == 0)
        def _zero_fix():
            zeros = jnp.zeros((L,), jnp.float32)

            def group_step(g, carry2):
                iv = idx_v[pl.ds(g * L, L)]
                for lane in range(L):
                    @pl.when(iv[lane] == 0)
                    def _zero_row(lane=lane):
                        r = g * L + lane
                        for j in range(D // L):
                            rows_v[r, pl.ds(j * L, L)] = zeros

                return carry2

            lax.fori_loop(0, CHUNK // L, group_step, 0)

        pltpu.sync_copy(rows_v, out_hbm.at[pl.ds(base, CHUNK)])
        return carry

    lax.fori_loop(0, N_CHUNKS, chunk_step, 0)


def kernel(inputs, embedding_weights):
    flat_idx = inputs.reshape(TOTAL)
    mesh = plsc.VectorSubcoreMesh(core_axis_name="c", subcore_axis_name="s")
    out = pl.kernel(
        _body,
        out_type=jax.ShapeDtypeStruct((TOTAL, D), jnp.float32),
        mesh=mesh,
        compiler_params=pltpu.CompilerParams(use_tc_tiling_on_sc=False),
        scratch_types=[
            pltpu.VMEM((CHUNK,), jnp.int32),
            pltpu.VMEM((CHUNK, D), jnp.float32),
            pltpu.SemaphoreType.DMA,
        ],
    )(flat_idx, embedding_weights)
    return out.reshape(inputs.shape + (D,))
